# baseline (speedup 1.0000x reference)
import functools

import jax
import jax.numpy as jnp
from jax import lax
from jax.experimental import pallas as pl
from jax.experimental.pallas import tpu as pltpu
from jax.experimental.pallas import tpu_sc as plsc

N = 10000
E = 160000
D_FEAT = 128
D_EDGE = 16
DIM = 32
B = 256

NC = 2
NS = 16
NW = NC * NS
EPW = E // NW
NG = 4
GC = 10
CHR = 125
RPS = N // NS

@functools.lru_cache(maxsize=None)
def _sc_mesh():
    return plsc.VectorSubcoreMesh(
        core_axis_name="c", subcore_axis_name="s", num_cores=NC, num_subcores=NS)


@functools.lru_cache(maxsize=None)
def _gather_kernel():
    def body(table, src2, xj, idx_v, rows_v, sem, wsem0, wsem1):
        c = lax.axis_index("c")
        s = lax.axis_index("s")
        wid = s * NC + c
        pltpu.sync_copy(src2.at[wid], idx_v)
        wsems = (wsem0, wsem1)
        wdesc = [None, None]
        for g in range(NG):
            b = g % 2
            if wdesc[b] is not None:
                wdesc[b].wait()
            descs = []
            for j in range(GC):
                descs.append(pltpu.async_copy(
                    table.at[idx_v.at[g * GC + j]], rows_v.at[b, j], sem))
            for d in descs:
                d.wait()
            wdesc[b] = pltpu.async_copy(rows_v.at[b], xj.at[wid, g], wsems[b])
        wdesc[0].wait()
        wdesc[1].wait()

    return pl.kernel(
        body,
        out_type=jax.ShapeDtypeStruct((NW, NG, GC, CHR, DIM), jnp.float32),
        mesh=_sc_mesh(),
        scratch_types=[
            pltpu.VMEM((NG * GC, CHR), jnp.int32),
            pltpu.VMEM((2, GC, CHR, DIM), jnp.float32),
            pltpu.SemaphoreType.DMA,
            pltpu.SemaphoreType.DMA,
            pltpu.SemaphoreType.DMA,
        ],
        compiler_params=pltpu.CompilerParams(use_tc_tiling_on_sc=False),
    )


def _gather_sc(table, src2):
    return _gather_kernel()(table, src2)


def _scatter_pipeline(msg5, dst2, wid, msg_v, idx_v, acc_sh, lsem0, lsem1,
                      ssem, extra_add=None):
    pltpu.sync_copy(dst2.at[wid], idx_v)
    lsems = (lsem0, lsem1)
    ld = [None, None]
    ld[0] = pltpu.async_copy(msg5.at[wid, 0], msg_v.at[0], lsems[0])
    for g in range(NG):
        b = g % 2
        if g + 1 < NG:
            ld[1 - b] = pltpu.async_copy(
                msg5.at[wid, g + 1], msg_v.at[1 - b], lsems[1 - b])
        ld[b].wait()
        descs = []
        for j in range(GC):
            descs.append(pltpu.async_copy(
                msg_v.at[b, j], acc_sh.at[idx_v.at[g * GC + j]], ssem,
                add=True))
            if extra_add is not None:
                descs.append(extra_add(g, j, ssem))
        for d in descs:
            d.wait()


@functools.lru_cache(maxsize=None)
def _scatter_kernel():
    def body(msg5, dst2, z32, out, msg_v, idx_v, acc_sh, lsem0, lsem1, ssem):
        c = lax.axis_index("c")
        s = lax.axis_index("s")
        wid = s * NC + c
        pltpu.sync_copy(z32.at[pl.ds(s * RPS, RPS)],
                        acc_sh.at[pl.ds(s * RPS, RPS)])
        plsc.subcore_barrier()
        _scatter_pipeline(msg5, dst2, wid, msg_v, idx_v, acc_sh, lsem0, lsem1,
                          ssem)
        plsc.subcore_barrier()
        pltpu.sync_copy(acc_sh.at[pl.ds(s * RPS, RPS)],
                        out.at[c, pl.ds(s * RPS, RPS)])

    return pl.kernel(
        body,
        out_type=jax.ShapeDtypeStruct((NC, N, DIM), jnp.float32),
        mesh=_sc_mesh(),
        scratch_types=[
            pltpu.VMEM((2, GC, CHR, DIM), jnp.float32),
            pltpu.VMEM((NG * GC, CHR), jnp.int32),
            pltpu.VMEM_SHARED((N, DIM), jnp.float32),
            pltpu.SemaphoreType.DMA,
            pltpu.SemaphoreType.DMA,
            pltpu.SemaphoreType.DMA,
        ],
        compiler_params=pltpu.CompilerParams(use_tc_tiling_on_sc=False),
    )


def _scatter_sc(msg5, dst2, z32):
    return _scatter_kernel()(msg5, dst2, z32)


@functools.lru_cache(maxsize=None)
def _scatter_count_kernel():
    def body(msg5, dst2, z32, z8, ones8, out, cnt, msg_v, idx_v, ones_v,
             acc_sh, acc8_sh, lsem0, lsem1, ssem):
        c = lax.axis_index("c")
        s = lax.axis_index("s")
        wid = s * NC + c
        pltpu.sync_copy(ones8, ones_v)
        pltpu.sync_copy(z32.at[pl.ds(s * RPS, RPS)],
                        acc_sh.at[pl.ds(s * RPS, RPS)])
        pltpu.sync_copy(z8.at[pl.ds(s * RPS, RPS)],
                        acc8_sh.at[pl.ds(s * RPS, RPS)])
        plsc.subcore_barrier()

        def ones_add(g, j, sem):
            return pltpu.async_copy(
                ones_v, acc8_sh.at[idx_v.at[g * GC + j]], sem, add=True)

        _scatter_pipeline(msg5, dst2, wid, msg_v, idx_v, acc_sh, lsem0, lsem1,
                          ssem, extra_add=ones_add)
        plsc.subcore_barrier()
        pltpu.sync_copy(acc_sh.at[pl.ds(s * RPS, RPS)],
                        out.at[c, pl.ds(s * RPS, RPS)])
        pltpu.sync_copy(acc8_sh.at[pl.ds(s * RPS, RPS)],
                        cnt.at[c, pl.ds(s * RPS, RPS)])

    return pl.kernel(
        body,
        out_type=(jax.ShapeDtypeStruct((NC, N, DIM), jnp.float32),
                  jax.ShapeDtypeStruct((NC, N, 8), jnp.float32)),
        mesh=_sc_mesh(),
        scratch_types=[
            pltpu.VMEM((2, GC, CHR, DIM), jnp.float32),
            pltpu.VMEM((NG * GC, CHR), jnp.int32),
            pltpu.VMEM((CHR, 8), jnp.float32),
            pltpu.VMEM_SHARED((N, DIM), jnp.float32),
            pltpu.VMEM_SHARED((N, 8), jnp.float32),
            pltpu.SemaphoreType.DMA,
            pltpu.SemaphoreType.DMA,
            pltpu.SemaphoreType.DMA,
        ],
        compiler_params=pltpu.CompilerParams(use_tc_tiling_on_sc=False),
    )


def _scatter_count_sc(msg5, dst2, z32, z8, ones8):
    return _scatter_count_kernel()(msg5, dst2, z32, z8, ones8)


def _lin0_body(x_ref, w_ref, b_ref, o_ref):
    acc = jnp.dot(x_ref[...], w_ref[...], preferred_element_type=jnp.float32)
    o_ref[...] = jnp.maximum(acc + b_ref[...], 0.0)


def _lin0(x, w0t, b0):
    r = 1000
    return pl.pallas_call(
        _lin0_body,
        grid=(N // r,),
        in_specs=[
            pl.BlockSpec((r, D_FEAT), lambda i: (i, 0)),
            pl.BlockSpec((D_FEAT, DIM), lambda i: (0, 0)),
            pl.BlockSpec((1, DIM), lambda i: (0, 0)),
        ],
        out_specs=pl.BlockSpec((r, DIM), lambda i: (i, 0)),
        out_shape=jax.ShapeDtypeStruct((N, DIM), jnp.float32),
    )(x, w0t, b0)


def _edge_body(ef_ref, xj_ref, w1t_ref, b1_ref, w2t_ref, b2_ref, rsel_ref,
               msg_ref):
    h = jnp.dot(ef_ref[...], w1t_ref[...], preferred_element_type=jnp.float32)
    h = jnp.maximum(h + b1_ref[...], 0.0)
    w_e = jnp.dot(h.astype(jnp.bfloat16), w2t_ref[...],
                  preferred_element_type=jnp.float32)
    w_e = w_e + b2_ref[...]
    x = jnp.dot(xj_ref[...].astype(jnp.bfloat16), rsel_ref[...],
                preferred_element_type=jnp.float32)
    p = w_e * x
    q = p[:, 0:512] + p[:, 512:1024]
    q = q[:, 0:256] + q[:, 256:512]
    q = q[:, 0:128] + q[:, 128:256]
    q = q[:, 0:64] + q[:, 64:128]
    msg_ref[...] = q[:, 0:DIM] + q[:, DIM:2 * DIM]


def _edge_msg(ef, xj, w1t, b1, w2t_bf, b2, rsel):
    te = 1000
    return pl.pallas_call(
        _edge_body,
        grid=(E // te,),
        in_specs=[
            pl.BlockSpec((te, D_EDGE), lambda i: (i, 0)),
            pl.BlockSpec((te, DIM), lambda i: (i, 0)),
            pl.BlockSpec((D_EDGE, 128), lambda i: (0, 0)),
            pl.BlockSpec((1, 128), lambda i: (0, 0)),
            pl.BlockSpec((128, DIM * DIM), lambda i: (0, 0)),
            pl.BlockSpec((1, DIM * DIM), lambda i: (0, 0)),
            pl.BlockSpec((DIM, DIM * DIM), lambda i: (0, 0)),
        ],
        out_specs=pl.BlockSpec((te, DIM), lambda i: (i, 0)),
        out_shape=jax.ShapeDtypeStruct((E, DIM), jnp.float32),
    )(ef, xj, w1t, b1, w2t_bf, b2, rsel)


def _gru_body(p0_ref, p1_ref, c0_ref, c1_ref, h_ref, bc_ref, wih_ref, bih_ref,
              whh_ref, bhh_ref, o_ref):
    agg = p0_ref[...] + p1_ref[...]
    cnt = c0_ref[:, 0:1] + c1_ref[:, 0:1]
    denom = jnp.maximum(cnt, 1.0)
    m = jnp.maximum(agg / denom + bc_ref[...], 0.0)
    h = h_ref[...]
    gi = jnp.dot(m, wih_ref[...], preferred_element_type=jnp.float32) + bih_ref[...]
    gh = jnp.dot(h, whh_ref[...], preferred_element_type=jnp.float32) + bhh_ref[...]
    i_r, i_z, i_n = gi[:, 0:DIM], gi[:, DIM:2 * DIM], gi[:, 2 * DIM:3 * DIM]
    h_r, h_z, h_n = gh[:, 0:DIM], gh[:, DIM:2 * DIM], gh[:, 2 * DIM:3 * DIM]
    r = jax.nn.sigmoid(i_r + h_r)
    z = jax.nn.sigmoid(i_z + h_z)
    n = jnp.tanh(i_n + r * h_n)
    o_ref[...] = (1.0 - z) * n + z * h


def _gru_step(p0, p1, c0, c1, h, bc, wih_t, bih, whh_t, bhh):
    r = 1000
    return pl.pallas_call(
        _gru_body,
        grid=(N // r,),
        in_specs=[
            pl.BlockSpec((r, DIM), lambda i: (i, 0)),
            pl.BlockSpec((r, DIM), lambda i: (i, 0)),
            pl.BlockSpec((r, 8), lambda i: (i, 0)),
            pl.BlockSpec((r, 8), lambda i: (i, 0)),
            pl.BlockSpec((r, DIM), lambda i: (i, 0)),
            pl.BlockSpec((1, DIM), lambda i: (0, 0)),
            pl.BlockSpec((DIM, 3 * DIM), lambda i: (0, 0)),
            pl.BlockSpec((1, 3 * DIM), lambda i: (0, 0)),
            pl.BlockSpec((DIM, 3 * DIM), lambda i: (0, 0)),
            pl.BlockSpec((1, 3 * DIM), lambda i: (0, 0)),
        ],
        out_specs=pl.BlockSpec((r, DIM), lambda i: (i, 0)),
        out_shape=jax.ShapeDtypeStruct((N, DIM), jnp.float32),
    )(p0, p1, c0, c1, h, bc, wih_t, bih, whh_t, bhh)


_S2S_T = 2500


def _s2s_body(out_ref, gi_ref, wih_ref, whh_ref, bsum_ref, q_ref, e_ref):
    iota = lax.broadcasted_iota(jnp.int32, (1, B), 1)
    qs = jnp.zeros((B, 2 * DIM), jnp.float32)
    hx = jnp.zeros((B, DIM), jnp.float32)
    cx = jnp.zeros((B, DIM), jnp.float32)
    neg = jnp.float32(-jnp.inf)
    for _ in range(3):
        gates = (jnp.dot(qs, wih_ref[...], preferred_element_type=jnp.float32)
                 + jnp.dot(hx, whh_ref[...], preferred_element_type=jnp.float32)
                 + bsum_ref[...])
        i_g = jax.nn.sigmoid(gates[:, 0:DIM])
        f_g = jax.nn.sigmoid(gates[:, DIM:2 * DIM])
        g_g = jnp.tanh(gates[:, 2 * DIM:3 * DIM])
        o_g = jax.nn.sigmoid(gates[:, 3 * DIM:4 * DIM])
        cx = f_g * cx + i_g * g_g
        hx = o_g * jnp.tanh(cx)
        q = hx
        emax = jnp.full((1, B), neg, jnp.float32)
        for t in range(N // _S2S_T):
            sl = slice(t * _S2S_T, (t + 1) * _S2S_T)
            oh = (gi_ref[sl, :] == iota)
            ohf = oh.astype(jnp.float32)
            qx = jnp.dot(ohf, q, preferred_element_type=jnp.float32)
            e_t = jnp.sum(out_ref[sl, :] * qx, axis=1, keepdims=True)
            e_ref[sl, :] = e_t
            emax = jnp.maximum(
                emax, jnp.max(jnp.where(oh, e_t, neg), axis=0, keepdims=True))
        emax = jnp.where(jnp.isfinite(emax), emax, 0.0)
        asum = jnp.zeros((1, B), jnp.float32)
        racc = jnp.zeros((B, DIM), jnp.float32)
        for t in range(N // _S2S_T):
            sl = slice(t * _S2S_T, (t + 1) * _S2S_T)
            oh = (gi_ref[sl, :] == iota)
            ohf = oh.astype(jnp.float32)
            e_t = e_ref[sl, :]
            em_row = jnp.dot(ohf, emax.reshape(B, 1),
                             preferred_element_type=jnp.float32)
            a = jnp.exp(e_t - em_row)
            asum = asum + jnp.sum(ohf * a, axis=0, keepdims=True)
            racc = racc + lax.dot_general(
                ohf, a * out_ref[sl, :], (((0,), (0,)), ((), ())),
                preferred_element_type=jnp.float32)
        r_read = racc / (asum.reshape(B, 1) + 1e-16)
        qs = jnp.concatenate([q, r_read], axis=1)
    q_ref[...] = qs


def _set2set(out, gi2, wih_t, whh_t, bsum):
    return pl.pallas_call(
        _s2s_body,
        out_shape=jax.ShapeDtypeStruct((B, 2 * DIM), jnp.float32),
        scratch_shapes=[pltpu.VMEM((N, 1), jnp.float32)],
    )(out, gi2, wih_t, whh_t, bsum)


def kernel(node_features, edge_index, edge_features, graph_index, W0, b0, W1,
           b1, W2, b2, b_conv, gru_Wih, gru_Whh, gru_bih, gru_bhh, ls_Wih,
           ls_Whh, ls_bih, ls_bhh):
    src2 = edge_index[0].reshape(NW, NG * GC, CHR)
    dst2 = edge_index[1].reshape(NW, NG * GC, CHR)
    gi2 = graph_index.reshape(N, 1)

    w0t = W0.T
    w1t = W1.T
    w2t_bf = W2.T.astype(jnp.bfloat16)
    wih_t = gru_Wih.T
    whh_t = gru_Whh.T
    ls_wih_t = ls_Wih.T
    ls_whh_t = ls_Whh.T
    rsel = (lax.broadcasted_iota(jnp.int32, (DIM, DIM * DIM), 1) // DIM
            == lax.broadcasted_iota(jnp.int32, (DIM, DIM * DIM), 0)
            ).astype(jnp.bfloat16)
    z32 = jnp.zeros((N, DIM), jnp.float32)
    z8 = jnp.zeros((N, 8), jnp.float32)
    ones8 = jnp.ones((CHR, 8), jnp.float32)

    out = _lin0(node_features, w0t, b0.reshape(1, DIM))
    h = out

    cnt = None
    for it in range(3):
        xj = _gather_sc(out, src2)
        msg = _edge_msg(edge_features, xj.reshape(E, DIM), w1t,
                        b1.reshape(1, 128), w2t_bf, b2.reshape(1, DIM * DIM),
                        rsel)
        msg5 = msg.reshape(NW, NG, GC, CHR, DIM)
        if it == 0:
            part, cnt = _scatter_count_sc(msg5, dst2, z32, z8, ones8)
        else:
            part = _scatter_sc(msg5, dst2, z32)
        h = _gru_step(part[0], part[1], cnt[0], cnt[1], h,
                      b_conv.reshape(1, DIM), wih_t,
                      gru_bih.reshape(1, 3 * DIM), whh_t,
                      gru_bhh.reshape(1, 3 * DIM))
        out = h

    q_star = _set2set(out, gi2, ls_wih_t, ls_whh_t,
                      (ls_bih + ls_bhh).reshape(1, 4 * DIM))
    return q_star, out


# speedup vs baseline: 1.0001x; 1.0001x over previous
import functools

import jax
import jax.numpy as jnp
from jax import lax
from jax.experimental import pallas as pl
from jax.experimental.pallas import tpu as pltpu
from jax.experimental.pallas import tpu_sc as plsc

N = 10000
E = 160000
D_FEAT = 128
D_EDGE = 16
DIM = 32
B = 256

NC = 2
NS = 16
NW = NC * NS
EPW = E // NW
NG = 4
GC = 10
CHR = 125
RPS = N // NS
GCR = GC * CHR

@functools.lru_cache(maxsize=None)
def _sc_mesh():
    return plsc.VectorSubcoreMesh(
        core_axis_name="c", subcore_axis_name="s", num_cores=NC, num_subcores=NS)


@functools.lru_cache(maxsize=None)
def _gather_kernel():
    def body(table, src2, xj, idx_v, rows_v, sem, wsem0, wsem1):
        c = lax.axis_index("c")
        s = lax.axis_index("s")
        wid = s * NC + c
        pltpu.sync_copy(src2.at[wid], idx_v)
        wsems = (wsem0, wsem1)
        wdesc = [None, None]
        for g in range(NG):
            b = g % 2
            if wdesc[b] is not None:
                wdesc[b].wait()
            descs = []
            for j in range(GC):
                descs.append(pltpu.async_copy(
                    table.at[idx_v.at[g * GC + j]],
                    rows_v.at[b, pl.ds(j * CHR, CHR)], sem))
            for d in descs:
                d.wait()
            wdesc[b] = pltpu.async_copy(
                rows_v.at[b], xj.at[pl.ds(wid * EPW + g * GCR, GCR)], wsems[b])
        wdesc[0].wait()
        wdesc[1].wait()

    return pl.kernel(
        body,
        out_type=jax.ShapeDtypeStruct((E, DIM), jnp.float32),
        mesh=_sc_mesh(),
        scratch_types=[
            pltpu.VMEM((NG * GC, CHR), jnp.int32),
            pltpu.VMEM((2, GCR, DIM), jnp.float32),
            pltpu.SemaphoreType.DMA,
            pltpu.SemaphoreType.DMA,
            pltpu.SemaphoreType.DMA,
        ],
        compiler_params=pltpu.CompilerParams(use_tc_tiling_on_sc=False),
    )


def _gather_sc(table, src2):
    return _gather_kernel()(table, src2)


def _scatter_pipeline(msg5, dst2, wid, msg_v, idx_v, acc_sh, lsem0, lsem1,
                      ssem, extra_add=None):
    pltpu.sync_copy(dst2.at[wid], idx_v)
    lsems = (lsem0, lsem1)
    ld = [None, None]
    ld[0] = pltpu.async_copy(
        msg5.at[pl.ds(wid * EPW, GCR)], msg_v.at[0], lsems[0])
    for g in range(NG):
        b = g % 2
        if g + 1 < NG:
            ld[1 - b] = pltpu.async_copy(
                msg5.at[pl.ds(wid * EPW + (g + 1) * GCR, GCR)],
                msg_v.at[1 - b], lsems[1 - b])
        ld[b].wait()
        descs = []
        for j in range(GC):
            descs.append(pltpu.async_copy(
                msg_v.at[b, pl.ds(j * CHR, CHR)],
                acc_sh.at[idx_v.at[g * GC + j]], ssem, add=True))
            if extra_add is not None:
                descs.append(extra_add(g, j, ssem))
        for d in descs:
            d.wait()


@functools.lru_cache(maxsize=None)
def _scatter_kernel():
    def body(msg5, dst2, z32, out, msg_v, idx_v, acc_sh, lsem0, lsem1, ssem):
        c = lax.axis_index("c")
        s = lax.axis_index("s")
        wid = s * NC + c
        pltpu.sync_copy(z32.at[pl.ds(s * RPS, RPS)],
                        acc_sh.at[pl.ds(s * RPS, RPS)])
        plsc.subcore_barrier()
        _scatter_pipeline(msg5, dst2, wid, msg_v, idx_v, acc_sh, lsem0, lsem1,
                          ssem)
        plsc.subcore_barrier()
        pltpu.sync_copy(acc_sh.at[pl.ds(s * RPS, RPS)],
                        out.at[c, pl.ds(s * RPS, RPS)])

    return pl.kernel(
        body,
        out_type=jax.ShapeDtypeStruct((NC, N, DIM), jnp.float32),
        mesh=_sc_mesh(),
        scratch_types=[
            pltpu.VMEM((2, GCR, DIM), jnp.float32),
            pltpu.VMEM((NG * GC, CHR), jnp.int32),
            pltpu.VMEM_SHARED((N, DIM), jnp.float32),
            pltpu.SemaphoreType.DMA,
            pltpu.SemaphoreType.DMA,
            pltpu.SemaphoreType.DMA,
        ],
        compiler_params=pltpu.CompilerParams(use_tc_tiling_on_sc=False),
    )


def _scatter_sc(msg5, dst2, z32):
    return _scatter_kernel()(msg5, dst2, z32)


@functools.lru_cache(maxsize=None)
def _scatter_count_kernel():
    def body(msg5, dst2, z32, z8, ones8, out, cnt, msg_v, idx_v, ones_v,
             acc_sh, acc8_sh, lsem0, lsem1, ssem):
        c = lax.axis_index("c")
        s = lax.axis_index("s")
        wid = s * NC + c
        pltpu.sync_copy(ones8, ones_v)
        pltpu.sync_copy(z32.at[pl.ds(s * RPS, RPS)],
                        acc_sh.at[pl.ds(s * RPS, RPS)])
        pltpu.sync_copy(z8.at[pl.ds(s * RPS, RPS)],
                        acc8_sh.at[pl.ds(s * RPS, RPS)])
        plsc.subcore_barrier()

        def ones_add(g, j, sem):
            return pltpu.async_copy(
                ones_v, acc8_sh.at[idx_v.at[g * GC + j]], sem, add=True)

        _scatter_pipeline(msg5, dst2, wid, msg_v, idx_v, acc_sh, lsem0, lsem1,
                          ssem, extra_add=ones_add)
        plsc.subcore_barrier()
        pltpu.sync_copy(acc_sh.at[pl.ds(s * RPS, RPS)],
                        out.at[c, pl.ds(s * RPS, RPS)])
        pltpu.sync_copy(acc8_sh.at[pl.ds(s * RPS, RPS)],
                        cnt.at[c, pl.ds(s * RPS, RPS)])

    return pl.kernel(
        body,
        out_type=(jax.ShapeDtypeStruct((NC, N, DIM), jnp.float32),
                  jax.ShapeDtypeStruct((NC, N, 8), jnp.float32)),
        mesh=_sc_mesh(),
        scratch_types=[
            pltpu.VMEM((2, GCR, DIM), jnp.float32),
            pltpu.VMEM((NG * GC, CHR), jnp.int32),
            pltpu.VMEM((CHR, 8), jnp.float32),
            pltpu.VMEM_SHARED((N, DIM), jnp.float32),
            pltpu.VMEM_SHARED((N, 8), jnp.float32),
            pltpu.SemaphoreType.DMA,
            pltpu.SemaphoreType.DMA,
            pltpu.SemaphoreType.DMA,
        ],
        compiler_params=pltpu.CompilerParams(use_tc_tiling_on_sc=False),
    )


def _scatter_count_sc(msg5, dst2, z32, z8, ones8):
    return _scatter_count_kernel()(msg5, dst2, z32, z8, ones8)


def _lin0_body(x_ref, w_ref, b_ref, o_ref):
    acc = jnp.dot(x_ref[...], w_ref[...], preferred_element_type=jnp.float32)
    o_ref[...] = jnp.maximum(acc + b_ref[...], 0.0)


def _lin0(x, w0t, b0):
    r = 1000
    return pl.pallas_call(
        _lin0_body,
        grid=(N // r,),
        in_specs=[
            pl.BlockSpec((r, D_FEAT), lambda i: (i, 0)),
            pl.BlockSpec((D_FEAT, DIM), lambda i: (0, 0)),
            pl.BlockSpec((1, DIM), lambda i: (0, 0)),
        ],
        out_specs=pl.BlockSpec((r, DIM), lambda i: (i, 0)),
        out_shape=jax.ShapeDtypeStruct((N, DIM), jnp.float32),
    )(x, w0t, b0)


def _edge_body(ef_ref, xj_ref, w1t_ref, b1_ref, w2t_ref, b2_ref, rsel_ref,
               msg_ref):
    h = jnp.dot(ef_ref[...], w1t_ref[...], preferred_element_type=jnp.float32)
    h = jnp.maximum(h + b1_ref[...], 0.0)
    w_e = jnp.dot(h.astype(jnp.bfloat16), w2t_ref[...],
                  preferred_element_type=jnp.float32)
    w_e = w_e + b2_ref[...]
    x = jnp.dot(xj_ref[...].astype(jnp.bfloat16), rsel_ref[...],
                preferred_element_type=jnp.float32)
    p = w_e * x
    q = p[:, 0:512] + p[:, 512:1024]
    q = q[:, 0:256] + q[:, 256:512]
    q = q[:, 0:128] + q[:, 128:256]
    q = q[:, 0:64] + q[:, 64:128]
    msg_ref[...] = q[:, 0:DIM] + q[:, DIM:2 * DIM]


def _edge_msg(ef, xj, w1t, b1, w2t_bf, b2, rsel):
    te = 1000
    return pl.pallas_call(
        _edge_body,
        grid=(E // te,),
        in_specs=[
            pl.BlockSpec((te, D_EDGE), lambda i: (i, 0)),
            pl.BlockSpec((te, DIM), lambda i: (i, 0)),
            pl.BlockSpec((D_EDGE, 128), lambda i: (0, 0)),
            pl.BlockSpec((1, 128), lambda i: (0, 0)),
            pl.BlockSpec((128, DIM * DIM), lambda i: (0, 0)),
            pl.BlockSpec((1, DIM * DIM), lambda i: (0, 0)),
            pl.BlockSpec((DIM, DIM * DIM), lambda i: (0, 0)),
        ],
        out_specs=pl.BlockSpec((te, DIM), lambda i: (i, 0)),
        out_shape=jax.ShapeDtypeStruct((E, DIM), jnp.float32),
    )(ef, xj, w1t, b1, w2t_bf, b2, rsel)


def _gru_body(p0_ref, p1_ref, c0_ref, c1_ref, h_ref, bc_ref, wih_ref, bih_ref,
              whh_ref, bhh_ref, o_ref):
    agg = p0_ref[...] + p1_ref[...]
    cnt = c0_ref[:, 0:1] + c1_ref[:, 0:1]
    denom = jnp.maximum(cnt, 1.0)
    m = jnp.maximum(agg / denom + bc_ref[...], 0.0)
    h = h_ref[...]
    gi = jnp.dot(m, wih_ref[...], preferred_element_type=jnp.float32) + bih_ref[...]
    gh = jnp.dot(h, whh_ref[...], preferred_element_type=jnp.float32) + bhh_ref[...]
    i_r, i_z, i_n = gi[:, 0:DIM], gi[:, DIM:2 * DIM], gi[:, 2 * DIM:3 * DIM]
    h_r, h_z, h_n = gh[:, 0:DIM], gh[:, DIM:2 * DIM], gh[:, 2 * DIM:3 * DIM]
    r = jax.nn.sigmoid(i_r + h_r)
    z = jax.nn.sigmoid(i_z + h_z)
    n = jnp.tanh(i_n + r * h_n)
    o_ref[...] = (1.0 - z) * n + z * h


def _gru_step(p0, p1, c0, c1, h, bc, wih_t, bih, whh_t, bhh):
    r = 1000
    return pl.pallas_call(
        _gru_body,
        grid=(N // r,),
        in_specs=[
            pl.BlockSpec((r, DIM), lambda i: (i, 0)),
            pl.BlockSpec((r, DIM), lambda i: (i, 0)),
            pl.BlockSpec((r, 8), lambda i: (i, 0)),
            pl.BlockSpec((r, 8), lambda i: (i, 0)),
            pl.BlockSpec((r, DIM), lambda i: (i, 0)),
            pl.BlockSpec((1, DIM), lambda i: (0, 0)),
            pl.BlockSpec((DIM, 3 * DIM), lambda i: (0, 0)),
            pl.BlockSpec((1, 3 * DIM), lambda i: (0, 0)),
            pl.BlockSpec((DIM, 3 * DIM), lambda i: (0, 0)),
            pl.BlockSpec((1, 3 * DIM), lambda i: (0, 0)),
        ],
        out_specs=pl.BlockSpec((r, DIM), lambda i: (i, 0)),
        out_shape=jax.ShapeDtypeStruct((N, DIM), jnp.float32),
    )(p0, p1, c0, c1, h, bc, wih_t, bih, whh_t, bhh)


_S2S_T = 2500


def _s2s_body(out_ref, gi_ref, wih_ref, whh_ref, bsum_ref, q_ref, e_ref):
    iota = lax.broadcasted_iota(jnp.int32, (1, B), 1)
    qs = jnp.zeros((B, 2 * DIM), jnp.float32)
    hx = jnp.zeros((B, DIM), jnp.float32)
    cx = jnp.zeros((B, DIM), jnp.float32)
    neg = jnp.float32(-jnp.inf)
    for _ in range(3):
        gates = (jnp.dot(qs, wih_ref[...], preferred_element_type=jnp.float32)
                 + jnp.dot(hx, whh_ref[...], preferred_element_type=jnp.float32)
                 + bsum_ref[...])
        i_g = jax.nn.sigmoid(gates[:, 0:DIM])
        f_g = jax.nn.sigmoid(gates[:, DIM:2 * DIM])
        g_g = jnp.tanh(gates[:, 2 * DIM:3 * DIM])
        o_g = jax.nn.sigmoid(gates[:, 3 * DIM:4 * DIM])
        cx = f_g * cx + i_g * g_g
        hx = o_g * jnp.tanh(cx)
        q = hx
        emax = jnp.full((1, B), neg, jnp.float32)
        for t in range(N // _S2S_T):
            sl = slice(t * _S2S_T, (t + 1) * _S2S_T)
            oh = (gi_ref[sl, :] == iota)
            ohf = oh.astype(jnp.float32)
            qx = jnp.dot(ohf, q, preferred_element_type=jnp.float32)
            e_t = jnp.sum(out_ref[sl, :] * qx, axis=1, keepdims=True)
            e_ref[sl, :] = e_t
            emax = jnp.maximum(
                emax, jnp.max(jnp.where(oh, e_t, neg), axis=0, keepdims=True))
        emax = jnp.where(jnp.isfinite(emax), emax, 0.0)
        asum = jnp.zeros((1, B), jnp.float32)
        racc = jnp.zeros((B, DIM), jnp.float32)
        for t in range(N // _S2S_T):
            sl = slice(t * _S2S_T, (t + 1) * _S2S_T)
            oh = (gi_ref[sl, :] == iota)
            ohf = oh.astype(jnp.float32)
            e_t = e_ref[sl, :]
            em_row = jnp.dot(ohf, emax.reshape(B, 1),
                             preferred_element_type=jnp.float32)
            a = jnp.exp(e_t - em_row)
            asum = asum + jnp.sum(ohf * a, axis=0, keepdims=True)
            racc = racc + lax.dot_general(
                ohf, a * out_ref[sl, :], (((0,), (0,)), ((), ())),
                preferred_element_type=jnp.float32)
        r_read = racc / (asum.reshape(B, 1) + 1e-16)
        qs = jnp.concatenate([q, r_read], axis=1)
    q_ref[...] = qs


def _set2set(out, gi2, wih_t, whh_t, bsum):
    return pl.pallas_call(
        _s2s_body,
        out_shape=jax.ShapeDtypeStruct((B, 2 * DIM), jnp.float32),
        scratch_shapes=[pltpu.VMEM((N, 1), jnp.float32)],
    )(out, gi2, wih_t, whh_t, bsum)


def kernel(node_features, edge_index, edge_features, graph_index, W0, b0, W1,
           b1, W2, b2, b_conv, gru_Wih, gru_Whh, gru_bih, gru_bhh, ls_Wih,
           ls_Whh, ls_bih, ls_bhh):
    src2 = edge_index[0].reshape(NW, NG * GC, CHR)
    dst2 = edge_index[1].reshape(NW, NG * GC, CHR)
    gi2 = graph_index.reshape(N, 1)

    w0t = W0.T
    w1t = W1.T
    w2t_bf = W2.T.astype(jnp.bfloat16)
    wih_t = gru_Wih.T
    whh_t = gru_Whh.T
    ls_wih_t = ls_Wih.T
    ls_whh_t = ls_Whh.T
    rsel = (lax.broadcasted_iota(jnp.int32, (DIM, DIM * DIM), 1) // DIM
            == lax.broadcasted_iota(jnp.int32, (DIM, DIM * DIM), 0)
            ).astype(jnp.bfloat16)
    z32 = jnp.zeros((N, DIM), jnp.float32)
    z8 = jnp.zeros((N, 8), jnp.float32)
    ones8 = jnp.ones((CHR, 8), jnp.float32)

    out = _lin0(node_features, w0t, b0.reshape(1, DIM))
    h = out

    cnt = None
    for it in range(3):
        xj = _gather_sc(out, src2)
        msg = _edge_msg(edge_features, xj, w1t,
                        b1.reshape(1, 128), w2t_bf, b2.reshape(1, DIM * DIM),
                        rsel)
        if it == 0:
            part, cnt = _scatter_count_sc(msg, dst2, z32, z8, ones8)
        else:
            part = _scatter_sc(msg, dst2, z32)
        h = _gru_step(part[0], part[1], cnt[0], cnt[1], h,
                      b_conv.reshape(1, DIM), wih_t,
                      gru_bih.reshape(1, 3 * DIM), whh_t,
                      gru_bhh.reshape(1, 3 * DIM))
        out = h

    q_star = _set2set(out, gi2, ls_wih_t, ls_whh_t,
                      (ls_bih + ls_bhh).reshape(1, 4 * DIM))
    return q_star, out


# speedup vs baseline: 1.1440x; 1.1439x over previous
import functools

import jax
import jax.numpy as jnp
from jax import lax
from jax.experimental import pallas as pl
from jax.experimental.pallas import tpu as pltpu
from jax.experimental.pallas import tpu_sc as plsc

N = 10000
E = 160000
D_FEAT = 128
D_EDGE = 16
DIM = 32
B = 256

NC = 2
NS = 16
NW = NC * NS
EPW = E // NW
NG = 20
GC = 2
CHR = 125
RPS = N // NS
GCR = GC * CHR
AW = 40
PW = 128

@functools.lru_cache(maxsize=None)
def _sc_mesh():
    return plsc.VectorSubcoreMesh(
        core_axis_name="c", subcore_axis_name="s", num_cores=NC, num_subcores=NS)


@functools.lru_cache(maxsize=None)
def _gather_kernel():
    def body(table, src2, xj, idx_v, rows_v, sem, wsem0, wsem1):
        c = lax.axis_index("c")
        s = lax.axis_index("s")
        wid = s * NC + c
        pltpu.sync_copy(src2.at[wid], idx_v)
        wsems = (wsem0, wsem1)
        wdesc = [None, None]
        for g in range(NG):
            b = g % 2
            if wdesc[b] is not None:
                wdesc[b].wait()
            descs = []
            for j in range(GC):
                descs.append(pltpu.async_copy(
                    table.at[idx_v.at[g * GC + j]],
                    rows_v.at[b, pl.ds(j * CHR, CHR)], sem))
            for d in descs:
                d.wait()
            wdesc[b] = pltpu.async_copy(
                rows_v.at[b], xj.at[pl.ds(wid * EPW + g * GCR, GCR)], wsems[b])
        wdesc[0].wait()
        wdesc[1].wait()

    return pl.kernel(
        body,
        out_type=jax.ShapeDtypeStruct((E, PW), jnp.float32),
        mesh=_sc_mesh(),
        scratch_types=[
            pltpu.VMEM((NG * GC, CHR), jnp.int32),
            pltpu.VMEM((2, GCR, PW), jnp.float32),
            pltpu.SemaphoreType.DMA,
            pltpu.SemaphoreType.DMA,
            pltpu.SemaphoreType.DMA,
        ],
        compiler_params=pltpu.CompilerParams(use_tc_tiling_on_sc=False),
    )


def _gather_sc(table, src2):
    return _gather_kernel()(table, src2)


def _scatter_pipeline(msg5, dst2, wid, msg_v, idx_v, acc_sh, lsem0, lsem1,
                      ssem, extra_add=None):
    pltpu.sync_copy(dst2.at[wid], idx_v)
    lsems = (lsem0, lsem1)
    ld = [None, None]
    ld[0] = pltpu.async_copy(
        msg5.at[pl.ds(wid * EPW, GCR), pl.ds(0, AW)], msg_v.at[0], lsems[0])
    for g in range(NG):
        b = g % 2
        if g + 1 < NG:
            ld[1 - b] = pltpu.async_copy(
                msg5.at[pl.ds(wid * EPW + (g + 1) * GCR, GCR), pl.ds(0, AW)],
                msg_v.at[1 - b], lsems[1 - b])
        ld[b].wait()
        descs = []
        for j in range(GC):
            descs.append(pltpu.async_copy(
                msg_v.at[b, pl.ds(j * CHR, CHR)],
                acc_sh.at[idx_v.at[g * GC + j]], ssem, add=True))
            if extra_add is not None:
                descs.append(extra_add(g, j, ssem))
        for d in descs:
            d.wait()


@functools.lru_cache(maxsize=None)
def _scatter_kernel():
    def body(msg5, dst2, zrows, out, msg_v, idx_v, acc_sh, lsem0, lsem1, ssem):
        c = lax.axis_index("c")
        s = lax.axis_index("s")
        wid = s * NC + c
        for k in range(RPS // CHR):
            pltpu.sync_copy(zrows,
                            acc_sh.at[pl.ds(s * RPS + k * CHR, CHR)])
        plsc.subcore_barrier()
        _scatter_pipeline(msg5, dst2, wid, msg_v, idx_v, acc_sh, lsem0, lsem1,
                          ssem)
        plsc.subcore_barrier()
        pltpu.sync_copy(acc_sh.at[pl.ds(s * RPS, RPS)],
                        out.at[c, pl.ds(s * RPS, RPS)])

    return pl.kernel(
        body,
        out_type=jax.ShapeDtypeStruct((NC, N, AW), jnp.float32),
        mesh=_sc_mesh(),
        scratch_types=[
            pltpu.VMEM((2, GCR, AW), jnp.float32),
            pltpu.VMEM((NG * GC, CHR), jnp.int32),
            pltpu.VMEM_SHARED((N, AW), jnp.float32),
            pltpu.SemaphoreType.DMA,
            pltpu.SemaphoreType.DMA,
            pltpu.SemaphoreType.DMA,
        ],
        compiler_params=pltpu.CompilerParams(use_tc_tiling_on_sc=False),
    )


def _scatter_sc(msg5, dst2, z32):
    return _scatter_kernel()(msg5, dst2, z32)


def _lin0_body(x_ref, w_ref, b_ref, o_ref):
    acc = jnp.dot(x_ref[...], w_ref[...], preferred_element_type=jnp.float32)
    r = acc.shape[0]
    o_ref[...] = jnp.concatenate(
        [jnp.maximum(acc + b_ref[...], 0.0),
         jnp.zeros((r, PW - DIM), jnp.float32)], axis=1)


def _lin0(x, w0t, b0):
    r = 1000
    return pl.pallas_call(
        _lin0_body,
        grid=(N // r,),
        in_specs=[
            pl.BlockSpec((r, D_FEAT), lambda i: (i, 0)),
            pl.BlockSpec((D_FEAT, DIM), lambda i: (0, 0)),
            pl.BlockSpec((1, DIM), lambda i: (0, 0)),
        ],
        out_specs=pl.BlockSpec((r, PW), lambda i: (i, 0)),
        out_shape=jax.ShapeDtypeStruct((N, PW), jnp.float32),
    )(x, w0t, b0)


def _edge_body(ef_ref, xj_ref, w1t_ref, b1_ref, w2t_ref, b2_ref, rsel_ref,
               msg_ref):
    h = jnp.dot(ef_ref[...], w1t_ref[...], preferred_element_type=jnp.float32)
    h = jnp.maximum(h + b1_ref[...], 0.0)
    w_e = jnp.dot(h.astype(jnp.bfloat16), w2t_ref[...],
                  preferred_element_type=jnp.float32)
    w_e = w_e + b2_ref[...]
    x = jnp.dot(xj_ref[...].astype(jnp.bfloat16), rsel_ref[...],
                preferred_element_type=jnp.float32)
    p = w_e * x
    q = p[:, 0:512] + p[:, 512:1024]
    q = q[:, 0:256] + q[:, 256:512]
    q = q[:, 0:128] + q[:, 128:256]
    q = q[:, 0:64] + q[:, 64:128]
    te = q.shape[0]
    msg_ref[...] = jnp.concatenate(
        [q[:, 0:DIM] + q[:, DIM:2 * DIM],
         jnp.ones((te, 1), jnp.float32),
         jnp.zeros((te, PW - DIM - 1), jnp.float32)], axis=1)


def _edge_msg(ef, xj, w1t, b1, w2t_bf, b2, rsel):
    te = 1000
    return pl.pallas_call(
        _edge_body,
        grid=(E // te,),
        in_specs=[
            pl.BlockSpec((te, D_EDGE), lambda i: (i, 0)),
            pl.BlockSpec((te, PW), lambda i: (i, 0)),
            pl.BlockSpec((D_EDGE, 128), lambda i: (0, 0)),
            pl.BlockSpec((1, 128), lambda i: (0, 0)),
            pl.BlockSpec((128, DIM * DIM), lambda i: (0, 0)),
            pl.BlockSpec((1, DIM * DIM), lambda i: (0, 0)),
            pl.BlockSpec((PW, DIM * DIM), lambda i: (0, 0)),
        ],
        out_specs=pl.BlockSpec((te, PW), lambda i: (i, 0)),
        out_shape=jax.ShapeDtypeStruct((E, PW), jnp.float32),
    )(ef, xj, w1t, b1, w2t_bf, b2, rsel)


def _gru_body(p0_ref, p1_ref, h_ref, bc_ref, wih_ref, bih_ref,
              whh_ref, bhh_ref, o_ref):
    psum = p0_ref[...] + p1_ref[...]
    agg = psum[:, 0:DIM]
    cnt = psum[:, DIM:DIM + 1]
    denom = jnp.maximum(cnt, 1.0)
    m = jnp.maximum(agg / denom + bc_ref[...], 0.0)
    h = h_ref[:, 0:DIM]
    gi = jnp.dot(m, wih_ref[...], preferred_element_type=jnp.float32) + bih_ref[...]
    gh = jnp.dot(h, whh_ref[...], preferred_element_type=jnp.float32) + bhh_ref[...]
    i_r, i_z, i_n = gi[:, 0:DIM], gi[:, DIM:2 * DIM], gi[:, 2 * DIM:3 * DIM]
    h_r, h_z, h_n = gh[:, 0:DIM], gh[:, DIM:2 * DIM], gh[:, 2 * DIM:3 * DIM]
    r = jax.nn.sigmoid(i_r + h_r)
    z = jax.nn.sigmoid(i_z + h_z)
    n = jnp.tanh(i_n + r * h_n)
    rr = h.shape[0]
    o_ref[...] = jnp.concatenate(
        [(1.0 - z) * n + z * h, jnp.zeros((rr, PW - DIM), jnp.float32)],
        axis=1)


def _gru_step(p0, p1, h, bc, wih_t, bih, whh_t, bhh):
    r = 1000
    return pl.pallas_call(
        _gru_body,
        grid=(N // r,),
        in_specs=[
            pl.BlockSpec((r, AW), lambda i: (i, 0)),
            pl.BlockSpec((r, AW), lambda i: (i, 0)),
            pl.BlockSpec((r, PW), lambda i: (i, 0)),
            pl.BlockSpec((1, DIM), lambda i: (0, 0)),
            pl.BlockSpec((DIM, 3 * DIM), lambda i: (0, 0)),
            pl.BlockSpec((1, 3 * DIM), lambda i: (0, 0)),
            pl.BlockSpec((DIM, 3 * DIM), lambda i: (0, 0)),
            pl.BlockSpec((1, 3 * DIM), lambda i: (0, 0)),
        ],
        out_specs=pl.BlockSpec((r, PW), lambda i: (i, 0)),
        out_shape=jax.ShapeDtypeStruct((N, PW), jnp.float32),
    )(p0, p1, h, bc, wih_t, bih, whh_t, bhh)


_S2S_T = 2500


def _s2s_body(out_ref, gi_ref, wih_ref, whh_ref, bsum_ref, q_ref, e_ref):
    iota = lax.broadcasted_iota(jnp.int32, (1, B), 1)
    qs = jnp.zeros((B, 2 * DIM), jnp.float32)
    hx = jnp.zeros((B, DIM), jnp.float32)
    cx = jnp.zeros((B, DIM), jnp.float32)
    neg = jnp.float32(-jnp.inf)
    for _ in range(3):
        gates = (jnp.dot(qs, wih_ref[...], preferred_element_type=jnp.float32)
                 + jnp.dot(hx, whh_ref[...], preferred_element_type=jnp.float32)
                 + bsum_ref[...])
        i_g = jax.nn.sigmoid(gates[:, 0:DIM])
        f_g = jax.nn.sigmoid(gates[:, DIM:2 * DIM])
        g_g = jnp.tanh(gates[:, 2 * DIM:3 * DIM])
        o_g = jax.nn.sigmoid(gates[:, 3 * DIM:4 * DIM])
        cx = f_g * cx + i_g * g_g
        hx = o_g * jnp.tanh(cx)
        q = hx
        emax = jnp.full((1, B), neg, jnp.float32)
        for t in range(N // _S2S_T):
            sl = slice(t * _S2S_T, (t + 1) * _S2S_T)
            oh = (gi_ref[sl, :] == iota)
            ohf = oh.astype(jnp.float32)
            qx = jnp.dot(ohf, q, preferred_element_type=jnp.float32)
            e_t = jnp.sum(out_ref[sl, 0:DIM] * qx, axis=1, keepdims=True)
            e_ref[sl, :] = e_t
            emax = jnp.maximum(
                emax, jnp.max(jnp.where(oh, e_t, neg), axis=0, keepdims=True))
        emax = jnp.where(jnp.isfinite(emax), emax, 0.0)
        asum = jnp.zeros((1, B), jnp.float32)
        racc = jnp.zeros((B, DIM), jnp.float32)
        for t in range(N // _S2S_T):
            sl = slice(t * _S2S_T, (t + 1) * _S2S_T)
            oh = (gi_ref[sl, :] == iota)
            ohf = oh.astype(jnp.float32)
            e_t = e_ref[sl, :]
            em_row = jnp.dot(ohf, emax.reshape(B, 1),
                             preferred_element_type=jnp.float32)
            a = jnp.exp(e_t - em_row)
            asum = asum + jnp.sum(ohf * a, axis=0, keepdims=True)
            racc = racc + lax.dot_general(
                ohf, a * out_ref[sl, 0:DIM], (((0,), (0,)), ((), ())),
                preferred_element_type=jnp.float32)
        r_read = racc / (asum.reshape(B, 1) + 1e-16)
        qs = jnp.concatenate([q, r_read], axis=1)
    q_ref[...] = qs


def _set2set(out, gi2, wih_t, whh_t, bsum):
    return pl.pallas_call(
        _s2s_body,
        out_shape=jax.ShapeDtypeStruct((B, 2 * DIM), jnp.float32),
        scratch_shapes=[pltpu.VMEM((N, 1), jnp.float32)],
    )(out, gi2, wih_t, whh_t, bsum)


def kernel(node_features, edge_index, edge_features, graph_index, W0, b0, W1,
           b1, W2, b2, b_conv, gru_Wih, gru_Whh, gru_bih, gru_bhh, ls_Wih,
           ls_Whh, ls_bih, ls_bhh):
    src2 = edge_index[0].reshape(NW, NG * GC, CHR)
    dst2 = edge_index[1].reshape(NW, NG * GC, CHR)
    gi2 = graph_index.reshape(N, 1)

    w0t = W0.T
    w1t = W1.T
    w2t_bf = W2.T.astype(jnp.bfloat16)
    wih_t = gru_Wih.T
    whh_t = gru_Whh.T
    ls_wih_t = ls_Wih.T
    ls_whh_t = ls_Whh.T
    rsel = (lax.broadcasted_iota(jnp.int32, (PW, DIM * DIM), 1) // DIM
            == lax.broadcasted_iota(jnp.int32, (PW, DIM * DIM), 0)
            ).astype(jnp.bfloat16)
    z128 = jnp.zeros((CHR, AW), jnp.float32)

    out = _lin0(node_features, w0t, b0.reshape(1, DIM))
    h = out

    for _ in range(3):
        xj = _gather_sc(out, src2)
        msg = _edge_msg(edge_features, xj, w1t,
                        b1.reshape(1, 128), w2t_bf, b2.reshape(1, DIM * DIM),
                        rsel)
        part = _scatter_sc(msg, dst2, z128)
        h = _gru_step(part[0], part[1], h,
                      b_conv.reshape(1, DIM), wih_t,
                      gru_bih.reshape(1, 3 * DIM), whh_t,
                      gru_bhh.reshape(1, 3 * DIM))
        out = h

    q_star = _set2set(out, gi2, ls_wih_t, ls_whh_t,
                      (ls_bih + ls_bhh).reshape(1, 4 * DIM))
    return q_star, out[:, 0:DIM]
